# baseline (speedup 1.0000x reference)
import functools

import jax
import jax.numpy as jnp
from jax import lax
from jax.experimental import pallas as pl
from jax.experimental.pallas import tpu as pltpu
from jax.experimental.pallas import tpu_sc as plsc

DIM = 256
K = 8192
B = 16
N = 1024
BN = B * N

TN = 512
GRID = BN // TN
NCH = 16


def _dist_argmax_body(xd_ref, et_ref, x2_ref, e2_ref, rev_ref,
                      dist_ref, idx_ref):
    x2 = x2_ref[...]
    xd = xd_ref[...]
    cw = K // NCH
    ms = []
    vals = []
    for c in range(NCH):
        xe2_c = lax.dot_general(
            xd, et_ref[:, c * cw:(c + 1) * cw],
            dimension_numbers=(((1,), (0,)), ((), ())),
            preferred_element_type=jnp.float32,
        )
        dist_c = (xe2_c - x2) - e2_ref[0, c * cw:(c + 1) * cw][None, :]
        dist_ref[:, c * cw:(c + 1) * cw] = dist_c
        m_c = jnp.max(dist_c, axis=1, keepdims=True)
        rev_c = rev_ref[0, c * cw:(c + 1) * cw][None, :]
        ms.append(m_c)
        vals.append(
            jnp.max(jnp.where(dist_c == m_c, rev_c, jnp.float32(0)),
                    axis=1, keepdims=True))
    m = functools.reduce(jnp.maximum, ms)
    val = functools.reduce(
        jnp.maximum,
        [jnp.where(m_c == m, v_c, jnp.float32(0))
         for m_c, v_c in zip(ms, vals)])
    idx_ref[0, 0, :] = (jnp.float32(K) - val[:, 0]).astype(jnp.int32)


def _dist_argmax(xd, embed_t, x2, e2):
    rev = (jnp.float32(K) - jnp.arange(K, dtype=jnp.float32)).reshape(1, K)
    return pl.pallas_call(
        _dist_argmax_body,
        grid=(GRID,),
        in_specs=[
            pl.BlockSpec((TN, DIM), lambda i: (i, 0)),
            pl.BlockSpec((DIM, K), lambda i: (0, 0)),
            pl.BlockSpec((TN, 1), lambda i: (i, 0)),
            pl.BlockSpec((1, K), lambda i: (0, 0)),
            pl.BlockSpec((1, K), lambda i: (0, 0)),
        ],
        out_specs=[
            pl.BlockSpec((TN, K), lambda i: (i, 0)),
            pl.BlockSpec((1, 1, TN), lambda i: (i, 0, 0)),
        ],
        out_shape=[
            jax.ShapeDtypeStruct((BN, K), jnp.float32),
            jax.ShapeDtypeStruct((GRID, 1, TN), jnp.int32),
        ],
    )(xd, embed_t, x2, e2, rev)


_CHUNK = 128


def _sc_gather(table, idx):
    info = plsc.get_sparse_core_info()
    nc, ns = info.num_cores, info.num_subcores
    b_per_w = BN // (nc * ns)
    nchunk = b_per_w // _CHUNK
    mesh = plsc.VectorSubcoreMesh(core_axis_name="c", subcore_axis_name="s")

    @functools.partial(
        pl.kernel,
        mesh=mesh,
        out_type=jax.ShapeDtypeStruct((BN, DIM), jnp.float32),
        scratch_types=[
            pltpu.VMEM((b_per_w,), jnp.int32),
            pltpu.VMEM((_CHUNK, DIM), jnp.float32),
            pltpu.SemaphoreType.DMA,
        ],
    )
    def gather_k(table_hbm, idx_hbm, out_hbm, idx_v, rows_v, sem):
        wid = lax.axis_index("s") * nc + lax.axis_index("c")
        base = wid * b_per_w
        pltpu.sync_copy(idx_hbm.at[pl.ds(base, b_per_w)], idx_v)
        for c in range(nchunk):
            pltpu.async_copy(
                table_hbm.at[idx_v.at[pl.ds(c * _CHUNK, _CHUNK)]], rows_v, sem
            ).wait()
            pltpu.sync_copy(rows_v, out_hbm.at[pl.ds(base + c * _CHUNK, _CHUNK)])

    return gather_k(table, idx)


def kernel(x, embed):
    flatten = x.reshape(1, BN, DIM)
    x2 = jnp.sum(flatten * flatten, axis=-1, keepdims=True)
    e2 = jnp.sum(embed * embed, axis=-1)
    x_flat = x.reshape(BN, DIM)
    embed2d = embed.reshape(K, DIM)
    dist, idx_blocks = _dist_argmax(
        x_flat + x_flat, embed2d.T, x2.reshape(BN, 1), e2.reshape(1, K))
    idx_flat = idx_blocks.reshape(BN)
    quantize = _sc_gather(embed2d, idx_flat).reshape(B, N, DIM)
    return quantize, idx_flat.reshape(B, N), dist.reshape(1, BN, K)


# speedup vs baseline: 1.1708x; 1.1708x over previous
import functools

import jax
import jax.numpy as jnp
from jax import lax
from jax.experimental import pallas as pl
from jax.experimental.pallas import tpu as pltpu
from jax.experimental.pallas import tpu_sc as plsc

DIM = 256
K = 8192
B = 16
N = 1024
BN = B * N

TN = 512
GRID = BN // TN
NCH = 16


def _dist_argmax_body(xb_ref, et_ref, e2_ref, rev_ref, dist_ref, idx_ref):
    xb = xb_ref[...]
    xd = xb + xb
    x2 = jnp.sum(xb * xb, axis=1, keepdims=True)
    cw = K // NCH
    ms = []
    for c in range(NCH):
        xe2_c = lax.dot_general(
            xd, et_ref[:, c * cw:(c + 1) * cw],
            dimension_numbers=(((1,), (0,)), ((), ())),
            preferred_element_type=jnp.float32,
        )
        dist_c = (xe2_c - x2) - e2_ref[0, c * cw:(c + 1) * cw][None, :]
        dist_ref[:, c * cw:(c + 1) * cw] = dist_c
        ms.append(jnp.max(dist_c, axis=1, keepdims=True))
    m = functools.reduce(jnp.maximum, ms)
    vals = []
    for c in range(NCH):
        d_c = dist_ref[:, c * cw:(c + 1) * cw]
        rev_c = rev_ref[0, c * cw:(c + 1) * cw][None, :]
        vals.append(jnp.max(jnp.where(d_c == m, rev_c, jnp.float32(0)), axis=1))
    val = functools.reduce(jnp.maximum, vals)
    idx_ref[0, 0, :] = (jnp.float32(K) - val).astype(jnp.int32)


def _dist_argmax(xb, embed_t, e2):
    rev = (jnp.float32(K) - jnp.arange(K, dtype=jnp.float32)).reshape(1, K)
    return pl.pallas_call(
        _dist_argmax_body,
        grid=(GRID,),
        in_specs=[
            pl.BlockSpec((TN, DIM), lambda i: (i, 0)),
            pl.BlockSpec((DIM, K), lambda i: (0, 0)),
            pl.BlockSpec((1, K), lambda i: (0, 0)),
            pl.BlockSpec((1, K), lambda i: (0, 0)),
        ],
        out_specs=[
            pl.BlockSpec((TN, K), lambda i: (i, 0)),
            pl.BlockSpec((1, 1, TN), lambda i: (i, 0, 0)),
        ],
        out_shape=[
            jax.ShapeDtypeStruct((BN, K), jnp.float32),
            jax.ShapeDtypeStruct((GRID, 1, TN), jnp.int32),
        ],
    )(xb, embed_t, e2, rev)


_CHUNK = 128


def _sc_gather(table, idx):
    info = plsc.get_sparse_core_info()
    nc, ns = info.num_cores, info.num_subcores
    b_per_w = BN // (nc * ns)
    nchunk = b_per_w // _CHUNK
    mesh = plsc.VectorSubcoreMesh(core_axis_name="c", subcore_axis_name="s")

    @functools.partial(
        pl.kernel,
        mesh=mesh,
        out_type=jax.ShapeDtypeStruct((BN, DIM), jnp.float32),
        scratch_types=[
            pltpu.VMEM((b_per_w,), jnp.int32),
            pltpu.VMEM((_CHUNK, DIM), jnp.float32),
            pltpu.SemaphoreType.DMA,
        ],
    )
    def gather_k(table_hbm, idx_hbm, out_hbm, idx_v, rows_v, sem):
        wid = lax.axis_index("s") * nc + lax.axis_index("c")
        base = wid * b_per_w
        pltpu.sync_copy(idx_hbm.at[pl.ds(base, b_per_w)], idx_v)
        for c in range(nchunk):
            pltpu.async_copy(
                table_hbm.at[idx_v.at[pl.ds(c * _CHUNK, _CHUNK)]], rows_v, sem
            ).wait()
            pltpu.sync_copy(rows_v, out_hbm.at[pl.ds(base + c * _CHUNK, _CHUNK)])

    return gather_k(table, idx)


def kernel(x, embed):
    e2 = jnp.sum(embed * embed, axis=-1)
    x_flat = x.reshape(BN, DIM)
    embed2d = embed.reshape(K, DIM)
    dist, idx_blocks = _dist_argmax(x_flat, embed2d.T, e2.reshape(1, K))
    idx_flat = idx_blocks.reshape(BN)
    quantize = _sc_gather(embed2d, idx_flat).reshape(B, N, DIM)
    return quantize, idx_flat.reshape(B, N), dist.reshape(1, BN, K)
